# baseline (speedup 1.0000x reference)
import functools

import jax
import jax.numpy as jnp
from jax import lax
from jax.experimental import pallas as pl
from jax.experimental.pallas import tpu as pltpu
from jax.experimental.pallas import tpu_sc as plsc

DIM = 64
PAD_DIM = 128
NC = 2
NS = 16
NW = NC * NS
SB = 256
SPAN = 2 * SB


def _emb_body(table_hbm, xt_hbm, out_hbm,
              ix0, ix1, a0, a1, b0, b1, gsem0, gsem1, osem0, osem1):
    wid = lax.axis_index("s") * NC + lax.axis_index("c")
    T = xt_hbm.shape[0]
    n_items = 2 * T
    s_base = wid * SPAN

    ix_v = (ix0, ix1)
    a_v = (a0, a1)
    b_v = (b0, b1)
    gsem = (gsem0, gsem1)
    osem = (osem0, osem1)

    iota16 = lax.iota(jnp.int32, 16)
    rows16 = [iota16 + 16 * sg for sg in range(SB // 16)]

    def idx_slice(par, h):
        return ix_v[par].at[pl.ds(h * SB, SB)]

    def out_slice(t_val, h):
        return out_hbm.at[t_val, :, pl.ds(s_base + h * SB, SB)]

    def fire_gather(t_val, par, h, slot):
        if h == 0:
            pltpu.sync_copy(xt_hbm.at[t_val, pl.ds(s_base, SPAN)], ix_v[par])
        pltpu.async_copy(table_hbm.at[idx_slice(par, h)], a_v[slot],
                         gsem[slot])

    def wait_gather(par, h, slot):
        pltpu.make_async_copy(table_hbm.at[idx_slice(par, h)], a_v[slot],
                              gsem[slot]).wait()

    def transpose(slot):
        a_ref, b_ref = a_v[slot], b_v[slot]

        def per_d(d, carry):
            col = jnp.full((16,), d, jnp.int32)
            for sg in range(SB // 16):
                v = plsc.load_gather(a_ref, [rows16[sg], col])
                b_ref[d, pl.ds(16 * sg, 16)] = v
            return carry

        lax.fori_loop(0, DIM, per_d, 0)

    fire_gather(0, 0, 0, 0)

    def outer(j, carry):
        for p in range(4):
            item = 4 * j + p
            t_val = 2 * j + p // 2
            par, h, slot = (p // 2) % 2, p % 2, p % 2
            npar, nh, nslot = ((p + 1) // 2) % 2, (p + 1) % 2, (p + 1) % 2
            nt_val = 2 * j + (p + 1) // 2

            @pl.when(item + 1 < n_items)
            def _fire_next():
                @pl.when(item >= 1)
                def _drain_prev_write():
                    pt_val = 2 * j + (p - 1) // 2
                    pltpu.make_async_copy(
                        b_v[nslot], out_slice(pt_val, (p - 1) % 2),
                        osem[nslot]).wait()
                fire_gather(nt_val, npar, nh, nslot)

            wait_gather(par, h, slot)
            transpose(slot)
            pltpu.async_copy(b_v[slot], out_slice(t_val, h), osem[slot])
        return carry

    lax.fori_loop(0, n_items // 4, outer, 0)

    pltpu.make_async_copy(b_v[0], out_slice(T - 1, 0), osem[0]).wait()
    pltpu.make_async_copy(b_v[1], out_slice(T - 1, 1), osem[1]).wait()


@jax.jit
def kernel(X, table):
    S, T = X.shape
    xt = X.T.astype(jnp.int32)
    table_p = jnp.pad(table, ((0, 0), (0, PAD_DIM - DIM)))
    mesh = plsc.VectorSubcoreMesh(core_axis_name="c", subcore_axis_name="s")
    k = functools.partial(
        pl.kernel,
        mesh=mesh,
        out_type=jax.ShapeDtypeStruct((T, DIM, S), jnp.float32),
        scratch_types=[
            pltpu.VMEM((SPAN,), jnp.int32),
            pltpu.VMEM((SPAN,), jnp.int32),
            pltpu.VMEM((SB, PAD_DIM), jnp.float32),
            pltpu.VMEM((SB, PAD_DIM), jnp.float32),
            pltpu.VMEM((DIM, SB), jnp.float32),
            pltpu.VMEM((DIM, SB), jnp.float32),
            pltpu.SemaphoreType.DMA,
            pltpu.SemaphoreType.DMA,
            pltpu.SemaphoreType.DMA,
            pltpu.SemaphoreType.DMA,
        ],
        compiler_params=pltpu.CompilerParams(
            use_tc_tiling_on_sc=True, needs_layout_passes=False),
    )(_emb_body)
    out_t = k(table_p, xt)
    return jnp.transpose(out_t, (2, 0, 1))


# speedup vs baseline: 1.7472x; 1.7472x over previous
import functools

import jax
import jax.numpy as jnp
from jax import lax
from jax.experimental import pallas as pl
from jax.experimental.pallas import tpu as pltpu
from jax.experimental.pallas import tpu_sc as plsc

DIM = 64
PAD_DIM = 128
NC = 2
NS = 16
NW = NC * NS
SB = 256
SPAN = 2 * SB


def _emb_body(table_hbm, xt_hbm, out_hbm,
              ix0, ix1, a0, a1, b0, b1, gsem0, gsem1, osem0, osem1):
    wid = lax.axis_index("s") * NC + lax.axis_index("c")
    T = xt_hbm.shape[0]
    n_items = 2 * T
    s_base = wid * SPAN

    ix_v = (ix0, ix1)
    a_v = (a0, a1)
    b_v = (b0, b1)
    gsem = (gsem0, gsem1)
    osem = (osem0, osem1)

    iota16 = lax.iota(jnp.int32, 16)
    rows16 = [iota16 + 16 * sg for sg in range(SB // 16)]

    def idx_slice(par, h):
        return ix_v[par].at[pl.ds(h * SB, SB)]

    def out_slice(t_val, h):
        return out_hbm.at[t_val, :, pl.ds(s_base + h * SB, SB)]

    def fire_gather(t_val, par, h, slot):
        if h == 0:
            pltpu.sync_copy(xt_hbm.at[t_val, pl.ds(s_base, SPAN)], ix_v[par])
        pltpu.async_copy(table_hbm.at[idx_slice(par, h)], a_v[slot],
                         gsem[slot])

    def wait_gather(par, h, slot):
        pltpu.make_async_copy(table_hbm.at[idx_slice(par, h)], a_v[slot],
                              gsem[slot]).wait()

    def transpose(slot):
        a_ref, b_ref = a_v[slot], b_v[slot]

        @plsc.parallel_loop(0, DIM, unroll=4)
        def per_d(d):
            col = jnp.full((16,), d, jnp.int32)
            for sg in range(SB // 16):
                v = plsc.load_gather(a_ref, [rows16[sg], col])
                b_ref[d, pl.ds(16 * sg, 16)] = v

    fire_gather(0, 0, 0, 0)

    def outer(j, carry):
        for p in range(4):
            item = 4 * j + p
            t_val = 2 * j + p // 2
            par, h, slot = (p // 2) % 2, p % 2, p % 2
            npar, nh, nslot = ((p + 1) // 2) % 2, (p + 1) % 2, (p + 1) % 2
            nt_val = 2 * j + (p + 1) // 2

            @pl.when(item + 1 < n_items)
            def _fire_next():
                @pl.when(item >= 1)
                def _drain_prev_write():
                    pt_val = 2 * j + (p - 1) // 2
                    pltpu.make_async_copy(
                        b_v[nslot], out_slice(pt_val, (p - 1) % 2),
                        osem[nslot]).wait()
                fire_gather(nt_val, npar, nh, nslot)

            wait_gather(par, h, slot)
            transpose(slot)
            pltpu.async_copy(b_v[slot], out_slice(t_val, h), osem[slot])
        return carry

    lax.fori_loop(0, n_items // 4, outer, 0)

    pltpu.make_async_copy(b_v[0], out_slice(T - 1, 0), osem[0]).wait()
    pltpu.make_async_copy(b_v[1], out_slice(T - 1, 1), osem[1]).wait()


@jax.jit
def kernel(X, table):
    S, T = X.shape
    xt = X.T.astype(jnp.int32)
    table_p = jnp.pad(table, ((0, 0), (0, PAD_DIM - DIM)))
    mesh = plsc.VectorSubcoreMesh(core_axis_name="c", subcore_axis_name="s")
    k = functools.partial(
        pl.kernel,
        mesh=mesh,
        out_type=jax.ShapeDtypeStruct((T, DIM, S), jnp.float32),
        scratch_types=[
            pltpu.VMEM((SPAN,), jnp.int32),
            pltpu.VMEM((SPAN,), jnp.int32),
            pltpu.VMEM((SB, PAD_DIM), jnp.float32),
            pltpu.VMEM((SB, PAD_DIM), jnp.float32),
            pltpu.VMEM((DIM, SB), jnp.float32),
            pltpu.VMEM((DIM, SB), jnp.float32),
            pltpu.SemaphoreType.DMA,
            pltpu.SemaphoreType.DMA,
            pltpu.SemaphoreType.DMA,
            pltpu.SemaphoreType.DMA,
        ],
        compiler_params=pltpu.CompilerParams(
            use_tc_tiling_on_sc=True, needs_layout_passes=False),
    )(_emb_body)
    out_t = k(table_p, xt)
    return jnp.transpose(out_t, (2, 0, 1))


# speedup vs baseline: 1.7666x; 1.0111x over previous
import functools

import jax
import jax.numpy as jnp
from jax import lax
from jax.experimental import pallas as pl
from jax.experimental.pallas import tpu as pltpu
from jax.experimental.pallas import tpu_sc as plsc

DIM = 64
PAD_DIM = 128
NC = 2
NS = 16
NW = NC * NS
SB = 256
SPAN = 2 * SB


def _emb_body(table_hbm, xt_hbm, out_hbm,
              ix0, ix1, a0, a1, b0, b1, gsem0, gsem1, osem0, osem1):
    wid = lax.axis_index("s") * NC + lax.axis_index("c")
    T = xt_hbm.shape[0]
    n_items = 2 * T
    s_base = wid * SPAN

    ix_v = (ix0, ix1)
    a_v = (a0, a1)
    b_v = (b0, b1)
    gsem = (gsem0, gsem1)
    osem = (osem0, osem1)

    iota16 = lax.iota(jnp.int32, 16)
    rows16 = [iota16 + 16 * sg for sg in range(SB // 16)]

    def idx_slice(par, h):
        return ix_v[par].at[pl.ds(h * SB, SB)]

    def out_slice(t_val, h):
        return out_hbm.at[t_val, :, pl.ds(s_base + h * SB, SB)]

    def fire_gather(t_val, par, h, slot):
        if h == 0:
            pltpu.sync_copy(xt_hbm.at[t_val, pl.ds(s_base, SPAN)], ix_v[par])
        pltpu.async_copy(table_hbm.at[idx_slice(par, h)], a_v[slot],
                         gsem[slot])

    def wait_gather(par, h, slot):
        pltpu.make_async_copy(table_hbm.at[idx_slice(par, h)], a_v[slot],
                              gsem[slot]).wait()

    def transpose(slot):
        a_ref, b_ref = a_v[slot], b_v[slot]

        @plsc.parallel_loop(0, DIM, unroll=8)
        def per_d(d):
            col = jnp.full((16,), d, jnp.int32)
            for sg in range(SB // 16):
                v = plsc.load_gather(a_ref, [rows16[sg], col])
                b_ref[d, pl.ds(16 * sg, 16)] = v

    fire_gather(0, 0, 0, 0)

    def outer(j, carry):
        for p in range(4):
            item = 4 * j + p
            t_val = 2 * j + p // 2
            par, h, slot = (p // 2) % 2, p % 2, p % 2
            npar, nh, nslot = ((p + 1) // 2) % 2, (p + 1) % 2, (p + 1) % 2
            nt_val = 2 * j + (p + 1) // 2

            @pl.when(item + 1 < n_items)
            def _fire_next():
                @pl.when(item >= 1)
                def _drain_prev_write():
                    pt_val = 2 * j + (p - 1) // 2
                    pltpu.make_async_copy(
                        b_v[nslot], out_slice(pt_val, (p - 1) % 2),
                        osem[nslot]).wait()
                fire_gather(nt_val, npar, nh, nslot)

            wait_gather(par, h, slot)
            transpose(slot)
            pltpu.async_copy(b_v[slot], out_slice(t_val, h), osem[slot])
        return carry

    lax.fori_loop(0, n_items // 4, outer, 0)

    pltpu.make_async_copy(b_v[0], out_slice(T - 1, 0), osem[0]).wait()
    pltpu.make_async_copy(b_v[1], out_slice(T - 1, 1), osem[1]).wait()


@jax.jit
def kernel(X, table):
    S, T = X.shape
    xt = X.T.astype(jnp.int32)
    table_p = jnp.pad(table, ((0, 0), (0, PAD_DIM - DIM)))
    mesh = plsc.VectorSubcoreMesh(core_axis_name="c", subcore_axis_name="s")
    k = functools.partial(
        pl.kernel,
        mesh=mesh,
        out_type=jax.ShapeDtypeStruct((T, DIM, S), jnp.float32),
        scratch_types=[
            pltpu.VMEM((SPAN,), jnp.int32),
            pltpu.VMEM((SPAN,), jnp.int32),
            pltpu.VMEM((SB, PAD_DIM), jnp.float32),
            pltpu.VMEM((SB, PAD_DIM), jnp.float32),
            pltpu.VMEM((DIM, SB), jnp.float32),
            pltpu.VMEM((DIM, SB), jnp.float32),
            pltpu.SemaphoreType.DMA,
            pltpu.SemaphoreType.DMA,
            pltpu.SemaphoreType.DMA,
            pltpu.SemaphoreType.DMA,
        ],
        compiler_params=pltpu.CompilerParams(
            use_tc_tiling_on_sc=True, needs_layout_passes=False),
    )(_emb_body)
    out_t = k(table_p, xt)
    return jnp.transpose(out_t, (2, 0, 1))


# speedup vs baseline: 5.0027x; 2.8318x over previous
import functools

import jax
import jax.numpy as jnp
from jax import lax
from jax.experimental import pallas as pl
from jax.experimental.pallas import tpu as pltpu
from jax.experimental.pallas import tpu_sc as plsc

DIM = 64
PAD_DIM = 128
NC = 2
NS = 16
NW = NC * NS
SB = 256
SPAN = 2 * SB


def _emb_body(table_hbm, xt_hbm, out_hbm,
              ix0, ix1, a0, a1, b0, b1, gsem0, gsem1, osem0, osem1):
    wid = lax.axis_index("s") * NC + lax.axis_index("c")
    T = xt_hbm.shape[0]
    n_items = 2 * T
    s_base = wid * SPAN

    ix_v = (ix0, ix1)
    a_v = (a0, a1)
    b_v = (b0, b1)
    gsem = (gsem0, gsem1)
    osem = (osem0, osem1)

    iota16 = lax.iota(jnp.int32, 16)
    rows16 = [iota16 + 16 * sg for sg in range(SB // 16)]
    skew16 = [lax.rem(iota16 + kk, 16) for kk in range(16)]

    def idx_slice(par, h):
        return ix_v[par].at[pl.ds(h * SB, SB)]

    def out_slice(t_val, h):
        return out_hbm.at[t_val, :, pl.ds(s_base + h * SB, SB)]

    def fire_gather(t_val, par, h, slot):
        if h == 0:
            pltpu.sync_copy(xt_hbm.at[t_val, pl.ds(s_base, SPAN)], ix_v[par])
        pltpu.async_copy(table_hbm.at[idx_slice(par, h)], a_v[slot],
                         gsem[slot])

    def wait_gather(par, h, slot):
        pltpu.make_async_copy(table_hbm.at[idx_slice(par, h)], a_v[slot],
                              gsem[slot]).wait()

    def transpose(slot):
        a_ref, b_ref = a_v[slot], b_v[slot]

        @plsc.parallel_loop(0, (DIM // 16) * (SB // 16), unroll=2)
        def per_block(bid):
            d0 = (bid & (DIM // 16 - 1)) * 16
            s0 = (bid >> 2) * 16
            rows = iota16 + s0
            for kk in range(16):
                drow = d0 + skew16[kk]
                v = plsc.load_gather(a_ref, [rows, drow])
                plsc.store_scatter(b_ref, [drow, rows], v)

    fire_gather(0, 0, 0, 0)

    def outer(j, carry):
        for p in range(4):
            item = 4 * j + p
            t_val = 2 * j + p // 2
            par, h, slot = (p // 2) % 2, p % 2, p % 2
            npar, nh, nslot = ((p + 1) // 2) % 2, (p + 1) % 2, (p + 1) % 2
            nt_val = 2 * j + (p + 1) // 2

            @pl.when(item + 1 < n_items)
            def _fire_next():
                @pl.when(item >= 1)
                def _drain_prev_write():
                    pt_val = 2 * j + (p - 1) // 2
                    pltpu.make_async_copy(
                        b_v[nslot], out_slice(pt_val, (p - 1) % 2),
                        osem[nslot]).wait()
                fire_gather(nt_val, npar, nh, nslot)

            wait_gather(par, h, slot)
            transpose(slot)
            pltpu.async_copy(b_v[slot], out_slice(t_val, h), osem[slot])
        return carry

    lax.fori_loop(0, n_items // 4, outer, 0)

    pltpu.make_async_copy(b_v[0], out_slice(T - 1, 0), osem[0]).wait()
    pltpu.make_async_copy(b_v[1], out_slice(T - 1, 1), osem[1]).wait()


@jax.jit
def kernel(X, table):
    S, T = X.shape
    xt = X.T.astype(jnp.int32)
    table_p = jnp.pad(table, ((0, 0), (0, PAD_DIM - DIM)))
    mesh = plsc.VectorSubcoreMesh(core_axis_name="c", subcore_axis_name="s")
    k = functools.partial(
        pl.kernel,
        mesh=mesh,
        out_type=jax.ShapeDtypeStruct((T, DIM, S), jnp.float32),
        scratch_types=[
            pltpu.VMEM((SPAN,), jnp.int32),
            pltpu.VMEM((SPAN,), jnp.int32),
            pltpu.VMEM((SB, PAD_DIM), jnp.float32),
            pltpu.VMEM((SB, PAD_DIM), jnp.float32),
            pltpu.VMEM((DIM, SB), jnp.float32),
            pltpu.VMEM((DIM, SB), jnp.float32),
            pltpu.SemaphoreType.DMA,
            pltpu.SemaphoreType.DMA,
            pltpu.SemaphoreType.DMA,
            pltpu.SemaphoreType.DMA,
        ],
        compiler_params=pltpu.CompilerParams(
            use_tc_tiling_on_sc=True, needs_layout_passes=False),
    )(_emb_body)
    out_t = k(table_p, xt)
    return jnp.transpose(out_t, (2, 0, 1))


# speedup vs baseline: 5.5131x; 1.1020x over previous
import functools

import jax
import jax.numpy as jnp
from jax import lax
from jax.experimental import pallas as pl
from jax.experimental.pallas import tpu as pltpu
from jax.experimental.pallas import tpu_sc as plsc

DIM = 64
PAD_DIM = 128
NC = 2
NS = 16
NW = NC * NS
SB = 256
SPAN = 2 * SB


def _emb_body(table_hbm, xt_hbm, out_hbm,
              ix0, ix1, a0, a1, b0, b1,
              gsem0, gsem1, osem0, osem1, isem0, isem1):
    wid = lax.axis_index("s") * NC + lax.axis_index("c")
    T = xt_hbm.shape[0]
    n_items = 2 * T
    s_base = wid * SPAN

    ix_v = (ix0, ix1)
    a_v = (a0, a1)
    b_v = (b0, b1)
    gsem = (gsem0, gsem1)
    osem = (osem0, osem1)
    isem = (isem0, isem1)

    iota16 = lax.iota(jnp.int32, 16)
    rows16 = [iota16 + 16 * sg for sg in range(SB // 16)]
    skew16 = [lax.rem(iota16 + kk, 16) for kk in range(16)]

    def idx_slice(par, h):
        return ix_v[par].at[pl.ds(h * SB, SB)]

    def out_slice(t_val, h):
        return out_hbm.at[t_val, :, pl.ds(s_base + h * SB, SB)]

    def fire_idx(t_val, par):
        pltpu.async_copy(xt_hbm.at[t_val, pl.ds(s_base, SPAN)], ix_v[par],
                         isem[par])

    def fire_gather(t_val, par, h, slot):
        if h == 0:
            pltpu.make_async_copy(xt_hbm.at[t_val, pl.ds(s_base, SPAN)],
                                  ix_v[par], isem[par]).wait()
        pltpu.async_copy(table_hbm.at[idx_slice(par, h)], a_v[slot],
                         gsem[slot])

    def wait_gather(par, h, slot):
        pltpu.make_async_copy(table_hbm.at[idx_slice(par, h)], a_v[slot],
                              gsem[slot]).wait()

    def transpose(slot):
        a_ref, b_ref = a_v[slot], b_v[slot]

        @plsc.parallel_loop(0, (DIM // 16) * (SB // 16), unroll=2)
        def per_block(bid):
            d0 = (bid & (DIM // 16 - 1)) * 16
            s0 = (bid >> 2) * 16
            rows = iota16 + s0
            for kk in range(16):
                drow = d0 + skew16[kk]
                v = plsc.load_gather(a_ref, [rows, drow])
                plsc.store_scatter(b_ref, [drow, rows], v)

    fire_idx(0, 0)
    fire_gather(0, 0, 0, 0)

    def outer(j, carry):
        for p in range(4):
            item = 4 * j + p
            t_val = 2 * j + p // 2
            par, h, slot = (p // 2) % 2, p % 2, p % 2
            npar, nh, nslot = ((p + 1) // 2) % 2, (p + 1) % 2, (p + 1) % 2
            nt_val = 2 * j + (p + 1) // 2

            @pl.when(item + 1 < n_items)
            def _fire_next():
                @pl.when(item >= 1)
                def _drain_prev_write():
                    pt_val = 2 * j + (p - 1) // 2
                    pltpu.make_async_copy(
                        b_v[nslot], out_slice(pt_val, (p - 1) % 2),
                        osem[nslot]).wait()
                fire_gather(nt_val, npar, nh, nslot)

            if p == 0:
                fire_idx(2 * j + 1, 1)
            elif p == 2:
                @pl.when(2 * j + 2 < T)
                def _prefetch_next_pair():
                    fire_idx(2 * j + 2, 0)

            wait_gather(par, h, slot)
            transpose(slot)
            pltpu.async_copy(b_v[slot], out_slice(t_val, h), osem[slot])
        return carry

    lax.fori_loop(0, n_items // 4, outer, 0)

    pltpu.make_async_copy(b_v[0], out_slice(T - 1, 0), osem[0]).wait()
    pltpu.make_async_copy(b_v[1], out_slice(T - 1, 1), osem[1]).wait()


@jax.jit
def kernel(X, table):
    S, T = X.shape
    xt = X.T.astype(jnp.int32)
    table_p = jnp.pad(table, ((0, 0), (0, PAD_DIM - DIM)))
    mesh = plsc.VectorSubcoreMesh(core_axis_name="c", subcore_axis_name="s")
    k = functools.partial(
        pl.kernel,
        mesh=mesh,
        out_type=jax.ShapeDtypeStruct((T, DIM, S), jnp.float32),
        scratch_types=[
            pltpu.VMEM((SPAN,), jnp.int32),
            pltpu.VMEM((SPAN,), jnp.int32),
            pltpu.VMEM((SB, PAD_DIM), jnp.float32),
            pltpu.VMEM((SB, PAD_DIM), jnp.float32),
            pltpu.VMEM((DIM, SB), jnp.float32),
            pltpu.VMEM((DIM, SB), jnp.float32),
            pltpu.SemaphoreType.DMA,
            pltpu.SemaphoreType.DMA,
            pltpu.SemaphoreType.DMA,
            pltpu.SemaphoreType.DMA,
            pltpu.SemaphoreType.DMA,
            pltpu.SemaphoreType.DMA,
        ],
        compiler_params=pltpu.CompilerParams(
            use_tc_tiling_on_sc=True, needs_layout_passes=False),
    )(_emb_body)
    out_t = k(table_p, xt)
    return jnp.transpose(out_t, (2, 0, 1))
